# baseline (speedup 1.0000x reference)
import functools

import jax
import jax.numpy as jnp
import numpy as np
from jax import lax
from jax.experimental import pallas as pl
from jax.experimental.pallas import tpu as pltpu
from jax.experimental.pallas import tpu_sc as plsc

NC = 2
NS = 16
C = 64


def _interleave_perm(d):
    p = []
    for k in range(d // 32):
        for u in range(16):
            p.append(32 * k + u)
            p.append(32 * k + 16 + u)
    return np.array(p, dtype=np.int32)


def _matmul_bf16(features, W_perm):
    n, d_in = features.shape
    d_out = W_perm.shape[1]
    bm = 1000

    def body(x_ref, w_ref, o_ref):
        o_ref[...] = jnp.dot(x_ref[...], w_ref[...],
                             preferred_element_type=jnp.float32
                             ).astype(jnp.bfloat16)

    return pl.pallas_call(
        body,
        grid=(n // bm,),
        in_specs=[
            pl.BlockSpec((bm, d_in), lambda i: (i, 0)),
            pl.BlockSpec((d_in, d_out), lambda i: (0, 0)),
        ],
        out_specs=pl.BlockSpec((bm, d_out), lambda i: (i, 0)),
        out_shape=jax.ShapeDtypeStruct((n, d_out), jnp.bfloat16),
    )(features, W_perm)


def _combine(partials):
    _, n, d = partials.shape
    bm = 1000

    def body(p_ref, o_ref):
        o_ref[...] = p_ref[0] + p_ref[1]

    return pl.pallas_call(
        body,
        grid=(n // bm,),
        in_specs=[pl.BlockSpec((2, bm, d), lambda i: (0, i, 0))],
        out_specs=pl.BlockSpec((bm, d), lambda i: (i, 0)),
        out_shape=jax.ShapeDtypeStruct((n, d), jnp.float32),
    )(partials)


def _sc_aggregate(h_bf16, packed3d, vals3d, zeros):
    n, d = zeros.shape
    chunks_per_tile = packed3d.shape[1]
    rows_per_tile = n // NS

    mesh = plsc.VectorSubcoreMesh(core_axis_name="c", subcore_axis_name="s")

    @functools.partial(
        pl.kernel,
        out_type=jax.ShapeDtypeStruct((NC, n, d), jnp.float32),
        mesh=mesh,
        scratch_types=[
            pltpu.VMEM((chunks_per_tile, C), jnp.int32),
            pltpu.VMEM((chunks_per_tile, C), jnp.float32),
            pltpu.VMEM((C, d), jnp.bfloat16),
            pltpu.VMEM((C, d), jnp.bfloat16),
            pltpu.VMEM((C, d), jnp.float32),
            pltpu.VMEM((C, d), jnp.float32),
            pltpu.VMEM((C,), jnp.int32),
            pltpu.VMEM((C,), jnp.int32),
            pltpu.VMEM((C,), jnp.int32),
            pltpu.VMEM((C,), jnp.int32),
            pltpu.VMEM_SHARED((n, d), jnp.float32),
            pltpu.SemaphoreType.DMA,
            pltpu.SemaphoreType.DMA,
            pltpu.SemaphoreType.DMA,
            pltpu.SemaphoreType.DMA,
            pltpu.SemaphoreType.DMA,
            pltpu.SemaphoreType.DMA,
        ],
        compiler_params=pltpu.CompilerParams(use_tc_tiling_on_sc=False,
                                             needs_layout_passes=False),
    )
    def kern(h_hbm, pck_hbm, vals_hbm, z_hbm, out_hbm,
             pckv, valv, gb0, gb1, sb0, sb1, cb0, cb1, rb0, rb1, acc,
             gsem0, gsem1, ssem0, ssem1, zsem, stsem):
        c = lax.axis_index("c")
        s = lax.axis_index("s")
        tile = c * NS + s
        gbufs, sbufs = (gb0, gb1), (sb0, sb1)
        cbufs, rbufs = (cb0, cb1), (rb0, rb1)
        gsems, ssems = (gsem0, gsem1), (ssem0, ssem1)

        sh16 = jnp.full((16,), 16, jnp.int32)
        mlow = jnp.full((16,), 0xFFFF, jnp.int32)
        mhigh = jnp.full((16,), -65536, jnp.int32)

        def unpack_cols(q, cb):
            for g in range(C // 16):
                pc = pckv[q, pl.ds(g * 16, 16)]
                cb[pl.ds(g * 16, 16)] = pc & mlow

        def unpack_rows(q, rb):
            for g in range(C // 16):
                pc = pckv[q, pl.ds(g * 16, 16)]
                rb[pl.ds(g * 16, 16)] = lax.shift_right_logical(pc, sh16)

        dz = pltpu.async_copy(
            z_hbm.at[pl.ds(s * rows_per_tile, rows_per_tile)],
            acc.at[pl.ds(s * rows_per_tile, rows_per_tile)], zsem)
        dp = pltpu.async_copy(pck_hbm.at[tile], pckv, stsem)
        dv = pltpu.async_copy(vals_hbm.at[tile], valv, stsem)
        dp.wait(); dv.wait()

        unpack_cols(0, cb0)
        unpack_cols(1, cb1)
        pltpu.async_copy(h_hbm.at[cb0], gb0, gsem0)
        pltpu.async_copy(h_hbm.at[cb1], gb1, gsem1)
        dz.wait()
        plsc.subcore_barrier()

        def wait_gather(b):
            pltpu.make_async_copy(h_hbm.at[cbufs[b]], gbufs[b],
                                  gsems[b]).wait()

        def wait_scatter(b):
            pltpu.make_async_copy(sbufs[b], acc.at[rbufs[b]],
                                  ssems[b]).wait()

        def scale(q, gbuf, sbuf):
            for g in range(C // 16):
                vv16 = valv[q, pl.ds(g * 16, 16)]
                for i16 in range(16):
                    lane = jnp.full((16,), i16, jnp.int32)
                    vv = vv16.at[lane].get(mode="promise_in_bounds")
                    i = g * 16 + i16
                    for k in range(d // 32):
                        v = plsc.bitcast(gbuf[i, pl.ds(k * 32, 32)],
                                         jnp.int32)
                        lo = plsc.bitcast(v << sh16, jnp.float32)
                        hi = plsc.bitcast(v & mhigh, jnp.float32)
                        sbuf[i, pl.ds(k * 32, 16)] = lo * vv
                        sbuf[i, pl.ds(k * 32 + 16, 16)] = hi * vv

        @pl.loop(0, chunks_per_tile - 2, step=2)
        def _pair(jo):
            for b in range(2):
                q = jo + b
                wait_gather(b)

                @pl.when(q >= 2)
                def _():
                    wait_scatter(b)

                scale(q, gbufs[b], sbufs[b])
                unpack_cols(q + 2, cbufs[b])
                pltpu.async_copy(h_hbm.at[cbufs[b]], gbufs[b], gsems[b])
                unpack_rows(q, rbufs[b])
                pltpu.async_copy(sbufs[b], acc.at[rbufs[b]], ssems[b],
                                 add=True)

        for b in range(2):
            q = chunks_per_tile - 2 + b
            wait_gather(b)
            wait_scatter(b)
            scale(q, gbufs[b], sbufs[b])
            unpack_rows(q, rbufs[b])
            pltpu.async_copy(sbufs[b], acc.at[rbufs[b]], ssems[b],
                             add=True)
        wait_scatter(0)
        wait_scatter(1)

        plsc.subcore_barrier()

        pltpu.sync_copy(acc.at[pl.ds(s * rows_per_tile, rows_per_tile)],
                        out_hbm.at[c, pl.ds(s * rows_per_tile, rows_per_tile)])

    return kern(h_bf16, packed3d, vals3d, zeros)


def kernel(features, edge_index, edge_vals, W):
    n, _ = features.shape
    d = W.shape[1]
    e = edge_vals.shape[0]

    nt = NC * NS
    cpt = -(-e // (nt * C))
    if cpt % 2:
        cpt += 1
    e_pad = nt * cpt * C

    rows = edge_index[0].astype(jnp.int32)
    cols = edge_index[1].astype(jnp.int32)
    vals = edge_vals.astype(jnp.float32)
    pad = e_pad - e
    if pad:
        rows = jnp.concatenate([rows, jnp.zeros((pad,), jnp.int32)])
        cols = jnp.concatenate([cols, jnp.zeros((pad,), jnp.int32)])
        vals = jnp.concatenate([vals, jnp.zeros((pad,), jnp.float32)])
    packed3d = ((rows << 16) | cols).reshape(nt, cpt, C)
    vals3d = vals.reshape(nt, cpt, C)
    zeros = jnp.zeros((n, d), jnp.float32)

    W_perm = W[:, jnp.asarray(_interleave_perm(d))]
    h_bf16 = _matmul_bf16(features, W_perm)
    partials = _sc_aggregate(h_bf16, packed3d, vals3d, zeros)
    return _combine(partials)


# speedup vs baseline: 1.1769x; 1.1769x over previous
import functools

import jax
import jax.numpy as jnp
from jax import lax
from jax.experimental import pallas as pl
from jax.experimental.pallas import tpu as pltpu
from jax.experimental.pallas import tpu_sc as plsc

NC = 2
NS = 16
C = 40


def _matmul(features, W):
    n, d_in = features.shape
    d_out = W.shape[1]
    bm = 1000

    def body(x_ref, w_ref, o_ref):
        o_ref[...] = jnp.dot(x_ref[...], w_ref[...],
                             preferred_element_type=jnp.float32)

    return pl.pallas_call(
        body,
        grid=(n // bm,),
        in_specs=[
            pl.BlockSpec((bm, d_in), lambda i: (i, 0)),
            pl.BlockSpec((d_in, d_out), lambda i: (0, 0)),
        ],
        out_specs=pl.BlockSpec((bm, d_out), lambda i: (i, 0)),
        out_shape=jax.ShapeDtypeStruct((n, d_out), jnp.float32),
    )(features, W)


def _combine(partials):
    _, n, d = partials.shape
    bm = 1000

    def body(p_ref, o_ref):
        o_ref[...] = p_ref[0] + p_ref[1]

    return pl.pallas_call(
        body,
        grid=(n // bm,),
        in_specs=[pl.BlockSpec((2, bm, d), lambda i: (0, i, 0))],
        out_specs=pl.BlockSpec((bm, d), lambda i: (i, 0)),
        out_shape=jax.ShapeDtypeStruct((n, d), jnp.float32),
    )(partials)


def _sc_aggregate(h, rows3d, cols3d, vals3d):
    n, d = h.shape
    chunks_per_tile = rows3d.shape[1]
    rows_per_tile = n // NS
    nd16 = d // 16

    mesh = plsc.VectorSubcoreMesh(core_axis_name="c", subcore_axis_name="s")

    @functools.partial(
        pl.kernel,
        out_type=jax.ShapeDtypeStruct((NC, n, d), jnp.float32),
        mesh=mesh,
        scratch_types=[
            pltpu.VMEM((chunks_per_tile, C), jnp.int32),
            pltpu.VMEM((chunks_per_tile, C), jnp.int32),
            pltpu.VMEM((chunks_per_tile, C), jnp.float32),
            pltpu.VMEM((C, d), jnp.float32),
            pltpu.VMEM((C, d), jnp.float32),
            pltpu.VMEM((C, d), jnp.float32),
            pltpu.VMEM((C, d), jnp.float32),
            pltpu.VMEM_SHARED((n, d), jnp.float32),
            pltpu.SemaphoreType.DMA,
            pltpu.SemaphoreType.DMA,
            pltpu.SemaphoreType.DMA,
            pltpu.SemaphoreType.DMA,
            pltpu.SemaphoreType.DMA,
            pltpu.SemaphoreType.DMA,
        ],
        compiler_params=pltpu.CompilerParams(use_tc_tiling_on_sc=False),
    )
    def kern(h_hbm, rows_hbm, cols_hbm, vals_hbm, out_hbm,
             rowv, colv, valv, gb0, gb1, sb0, sb1, acc,
             gsem0, gsem1, ssem0, ssem1, zsem, stsem):
        c = lax.axis_index("c")
        s = lax.axis_index("s")
        tile = c * NS + s
        gbufs, sbufs = (gb0, gb1), (sb0, sb1)
        gsems, ssems = (gsem0, gsem1), (ssem0, ssem1)

        dr = pltpu.async_copy(rows_hbm.at[tile], rowv, stsem)
        dc = pltpu.async_copy(cols_hbm.at[tile], colv, stsem)
        dv = pltpu.async_copy(vals_hbm.at[tile], valv, stsem)

        zv = jnp.zeros((16,), jnp.float32)
        for zb in (sb0, sb1):
            for i in range(C):
                for k in range(nd16):
                    zb[i, pl.ds(k * 16, 16)] = zv
        zdescs = []
        base = s * rows_per_tile
        off = 0
        while off < rows_per_tile:
            step = min(2 * C, rows_per_tile - off)
            if step > C:
                zdescs.append(pltpu.async_copy(
                    sb0, acc.at[pl.ds(base + off, C)], zsem))
                zdescs.append(pltpu.async_copy(
                    sb1.at[pl.ds(0, step - C)],
                    acc.at[pl.ds(base + off + C, step - C)], zsem))
            else:
                zdescs.append(pltpu.async_copy(
                    sb0.at[pl.ds(0, step)],
                    acc.at[pl.ds(base + off, step)], zsem))
            off += step

        dr.wait(); dc.wait(); dv.wait()

        pltpu.async_copy(h_hbm.at[colv.at[0]], gb0, gsem0)
        pltpu.async_copy(h_hbm.at[colv.at[1]], gb1, gsem1)
        for zd in zdescs:
            zd.wait()
        plsc.subcore_barrier()

        def wait_gather(b):
            pltpu.make_async_copy(h_hbm.at[colv.at[0]], gbufs[b],
                                  gsems[b]).wait()

        def wait_scatter(b):
            pltpu.make_async_copy(sbufs[b], acc.at[rowv.at[0]],
                                  ssems[b]).wait()

        def scale(q, gbuf, sbuf):
            groups = [(g * 16, 0) for g in range(C // 16)]
            if C % 16:
                groups.append((C - 16, 16 - C % 16))
            for base, start in groups:
                vv16 = valv[q, pl.ds(base, 16)]
                for i16 in range(start, 16):
                    lane = jnp.full((16,), i16, jnp.int32)
                    vv = vv16.at[lane].get(mode="promise_in_bounds")
                    i = base + i16
                    for k in range(nd16):
                        sl = pl.ds(k * 16, 16)
                        sbuf[i, sl] = gbuf[i, sl] * vv

        @pl.loop(0, chunks_per_tile - 2, step=2)
        def _pair(jo):
            for b in range(2):
                q = jo + b
                wait_gather(b)

                @pl.when(q >= 2)
                def _():
                    wait_scatter(b)

                scale(q, gbufs[b], sbufs[b])
                pltpu.async_copy(h_hbm.at[colv.at[q + 2]], gbufs[b],
                                 gsems[b])
                pltpu.async_copy(sbufs[b], acc.at[rowv.at[q]], ssems[b],
                                 add=True)

        for b in range(2):
            q = chunks_per_tile - 2 + b
            wait_gather(b)
            wait_scatter(b)
            scale(q, gbufs[b], sbufs[b])
            pltpu.async_copy(sbufs[b], acc.at[rowv.at[q]], ssems[b],
                             add=True)
        wait_scatter(0)
        wait_scatter(1)

        plsc.subcore_barrier()

        pltpu.sync_copy(acc.at[pl.ds(s * rows_per_tile, rows_per_tile)],
                        out_hbm.at[c, pl.ds(s * rows_per_tile, rows_per_tile)])

    return kern(h, rows3d, cols3d, vals3d)


def kernel(features, edge_index, edge_vals, W):
    n, _ = features.shape
    d = W.shape[1]
    e = edge_vals.shape[0]

    nt = NC * NS
    cpt = e // (nt * C)
    rows3d = edge_index[0].astype(jnp.int32).reshape(nt, cpt, C)
    cols3d = edge_index[1].astype(jnp.int32).reshape(nt, cpt, C)
    vals3d = edge_vals.reshape(nt, cpt, C)

    h = _matmul(features, W)
    partials = _sc_aggregate(h, rows3d, cols3d, vals3d)
    return _combine(partials)
